# baseline (speedup 1.0000x reference)
import jax
import jax.numpy as jnp
from jax import lax
from jax.experimental import pallas as pl
from jax.experimental.pallas import tpu as pltpu
from jax.experimental.pallas import tpu_sc as plsc

N_FIELDS = 26
VOCAB = 100000
EMB = 16
NUM_DENSE = 13
BATCH = 16384

ROWS_PER_LINE = 8
LINES_PER_FIELD = VOCAB // ROWS_PER_LINE

NW = 32
SPW = BATCH // NW
CH = 128
NCH = SPW // CH


def _sc_body(tab_hbm, xs_hbm, xd_hbm, w_hbm, out_hbm,
             xs_v, xd_v, idx_v, off_v, buf0, buf1, s_v, ss_v, w_v, t_v, out_v,
             sem):
    cid = lax.axis_index("c")
    sid = lax.axis_index("s")
    wid = sid * 2 + cid

    pltpu.sync_copy(w_hbm, w_v)
    half_wemb = w_v[0, :] * 0.5
    w_dense = w_v[1, :]
    bias_vec = w_v[2, :]
    lane = lax.iota(jnp.int32, 16)
    dcol = jnp.minimum(lane, NUM_DENSE - 1)
    bufs = (buf0, buf1)

    for c in range(NCH):
        sample0 = (wid * NCH + c) * CH

        pltpu.sync_copy(xs_hbm.at[pl.ds(sample0, CH), :], xs_v)
        pltpu.sync_copy(xd_hbm.at[pl.ds(sample0, CH), :], xd_v)

        def tr_body(g, _):
            rows = g * 16 + lane
            for f in range(N_FIELDS):
                ids = plsc.load_gather(xs_v, [rows, jnp.full((16,), f, jnp.int32)])
                idx_v[f, pl.ds(g * 16, 16)] = (
                    lax.shift_right_logical(ids, 3) + f * LINES_PER_FIELD)
                off_v[f, pl.ds(g * 16, 16)] = (ids & 7) * EMB
            return 0

        lax.fori_loop(0, CH // 16, tr_body, 0)

        cp = pltpu.async_copy(tab_hbm.at[idx_v.at[0]], bufs[0], sem)
        for f in range(N_FIELDS):
            cp.wait()
            if f + 1 < N_FIELDS:
                cp = pltpu.async_copy(
                    tab_hbm.at[idx_v.at[f + 1]], bufs[(f + 1) % 2], sem)
            buf = bufs[f % 2]

            if f == 0:
                def acc_body0(j, _):
                    jv = jnp.full((16,), j, jnp.int32)
                    off = plsc.load_gather(off_v, [jnp.zeros((16,), jnp.int32), jv])
                    e = plsc.load_gather(buf0, [jv, off + lane])
                    s_v[j, :] = e
                    ss_v[j, :] = e * e
                    return 0
                lax.fori_loop(0, CH, acc_body0, 0)
            else:
                def acc_body(j, _, f=f, buf=buf):
                    jv = jnp.full((16,), j, jnp.int32)
                    off = plsc.load_gather(off_v, [jnp.full((16,), f, jnp.int32), jv])
                    e = plsc.load_gather(buf, [jv, off + lane])
                    s_v[j, :] = s_v[j, :] + e
                    ss_v[j, :] = ss_v[j, :] + e * e
                    return 0
                lax.fori_loop(0, CH, acc_body, 0)

        def group_body(g, _):
            def lane_body(l, _):
                j = g * 16 + l
                s = s_v[j, :]
                ss = ss_v[j, :]
                d = plsc.load_gather(xd_v, [jnp.full((16,), j, jnp.int32), dcol])
                t_v[l, :] = (s * s - ss) * half_wemb + d * w_dense
                return 0

            lax.fori_loop(0, 16, lane_body, 0)
            red = plsc.load_gather(t_v, [lane, jnp.zeros((16,), jnp.int32)])
            for d in range(1, EMB):
                red = red + plsc.load_gather(
                    t_v, [lane, jnp.full((16,), d, jnp.int32)])
            logits = red + bias_vec
            out_v[pl.ds(g * 16, 16)] = 1.0 / (1.0 + jnp.exp(-logits))
            return 0

        lax.fori_loop(0, CH // 16, group_body, 0)
        pltpu.sync_copy(out_v, out_hbm.at[pl.ds(sample0, CH)])


@jax.jit
def kernel(X_sparse, X_dense, tables, dnn_w, dnn_b):
    tab_lines = tables.reshape(N_FIELDS * LINES_PER_FIELD, ROWS_PER_LINE * EMB)
    w_emb = dnn_w[:EMB, 0]
    w_den = jnp.pad(dnn_w[EMB:, 0], (0, EMB - NUM_DENSE))
    b16 = jnp.broadcast_to(dnn_b, (EMB,))
    wcat = jnp.stack([w_emb, w_den, b16])

    call = pl.kernel(
        _sc_body,
        out_type=jax.ShapeDtypeStruct((BATCH,), jnp.float32),
        mesh=plsc.VectorSubcoreMesh(core_axis_name="c", subcore_axis_name="s"),
        compiler_params=pltpu.CompilerParams(needs_layout_passes=False),
        scratch_types=[
            pltpu.VMEM((CH, N_FIELDS), jnp.int32),
            pltpu.VMEM((CH, NUM_DENSE), jnp.float32),
            pltpu.VMEM((N_FIELDS, CH), jnp.int32),
            pltpu.VMEM((N_FIELDS, CH), jnp.int32),
            pltpu.VMEM((CH, ROWS_PER_LINE * EMB), jnp.float32),
            pltpu.VMEM((CH, ROWS_PER_LINE * EMB), jnp.float32),
            pltpu.VMEM((CH, EMB), jnp.float32),
            pltpu.VMEM((CH, EMB), jnp.float32),
            pltpu.VMEM((3, EMB), jnp.float32),
            pltpu.VMEM((16, EMB), jnp.float32),
            pltpu.VMEM((CH,), jnp.float32),
            pltpu.SemaphoreType.DMA,
        ],
    )
    out = call(tab_lines, X_sparse.astype(jnp.int32), X_dense, wcat)
    return out.reshape(BATCH, 1)
